# baseline (speedup 1.0000x reference)
import jax
import jax.numpy as jnp
from jax import lax
from jax.experimental import pallas as pl
from jax.experimental.pallas import tpu as pltpu
from jax.experimental.pallas import tpu_sc as plsc

N = 10000
E = 320000
D = 128

NC = 2
NS = 16
NW = NC * NS
WIN = 64
CH = 4
NB = 4
EPW = E // NW
NITER = 20
NWIN = 8 * NITER
NCHUNK = 42
WALLOC = NCHUNK * CH
N_PAD = 10112
DUMP = N
RPT = N_PAD // NS


def _sc_agg_kernel():
    mesh = plsc.VectorSubcoreMesh(core_axis_name="c", subcore_axis_name="s")
    out = jax.ShapeDtypeStruct((NC, N_PAD, D), jnp.float32)
    scratch = [
        pltpu.VMEM_SHARED((N_PAD, D), jnp.float32),
        pltpu.VMEM((CH, 2, WIN), jnp.int32),
        pltpu.VMEM((CH, 2, WIN), jnp.int32),
        pltpu.VMEM((NB, WIN, D), jnp.float32),
        pltpu.SemaphoreType.DMA,
        pltpu.SemaphoreType.DMA,
        pltpu.SemaphoreType.DMA,
        pltpu.SemaphoreType.DMA,
        pltpu.SemaphoreType.DMA,
        pltpu.SemaphoreType.DMA,
        pltpu.SemaphoreType.DMA,
        pltpu.SemaphoreType.DMA,
        pltpu.SemaphoreType.DMA,
        pltpu.SemaphoreType.DMA,
    ]

    def body(x_hbm, idx_hbm, zrow_hbm, agg_out, acc, icba, icbb, rows,
             g0, g1, g2, g3, s0, s1, s2, s3, isema, isemb):
        c = lax.axis_index("c")
        s = lax.axis_index("s")
        wid = c * NS + s
        r0 = s * RPT
        gsem = (g0, g1, g2, g3)
        ssem = (s0, s1, s2, s3)

        def gstart(icb, i, b):
            pltpu.async_copy(x_hbm.at[icb.at[i, 0]], rows.at[b], gsem[b])

        def gwait(b):
            pltpu.make_async_copy(x_hbm.at[icba.at[0, 0]], rows.at[b],
                                  gsem[b]).wait()

        def sstart(icb, i, b):
            pltpu.async_copy(rows.at[b], acc.at[icb.at[i, 1]], ssem[b],
                             add=True)

        def swait(icb, i, b):
            pltpu.make_async_copy(rows.at[b], acc.at[icb.at[i, 1]],
                                  ssem[b]).wait()

        pltpu.sync_copy(zrow_hbm.at[pl.ds(r0, RPT)], acc.at[pl.ds(r0, RPT)])
        pltpu.sync_copy(idx_hbm.at[wid, 0], icba)
        pltpu.async_copy(idx_hbm.at[wid, 1], icbb, isemb)
        for b in range(NB):
            gstart(icba, b, b)
        plsc.subcore_barrier()

        @pl.loop(0, NITER)
        def _(k):
            ca = 2 * k
            for b in range(NB):
                gwait(b)
                sstart(icba, b, b)
            pltpu.make_async_copy(idx_hbm.at[wid, 0], icbb, isemb).wait()
            for b in range(NB):
                swait(icba, b, b)
                gstart(icbb, b, b)
            pltpu.async_copy(idx_hbm.at[wid, ca + 2], icba, isema)
            for b in range(NB):
                gwait(b)
                sstart(icbb, b, b)
            pltpu.make_async_copy(idx_hbm.at[wid, 0], icba, isema).wait()
            for b in range(NB):
                swait(icbb, b, b)
                gstart(icba, b, b)
            pltpu.async_copy(idx_hbm.at[wid, ca + 3], icbb, isemb)

        for b in range(NB):
            gwait(b)
        pltpu.make_async_copy(idx_hbm.at[wid, 0], icbb, isemb).wait()
        plsc.subcore_barrier()

        pltpu.sync_copy(acc.at[pl.ds(r0, RPT)], agg_out.at[c].at[pl.ds(r0, RPT)])

    return pl.kernel(body, out_type=out, mesh=mesh, scratch_types=scratch)


def _sc_cnt_kernel():
    mesh = plsc.VectorSubcoreMesh(core_axis_name="c", subcore_axis_name="s")
    out = jax.ShapeDtypeStruct((NC, N_PAD, D), jnp.float32)
    scratch = [
        pltpu.VMEM_SHARED((N_PAD, D), jnp.float32),
        pltpu.VMEM((CH, 2, WIN), jnp.int32),
        pltpu.VMEM((CH, 2, WIN), jnp.int32),
        pltpu.VMEM((WIN, D), jnp.float32),
        pltpu.SemaphoreType.DMA,
        pltpu.SemaphoreType.DMA,
    ]

    def body(idx_hbm, zrow_hbm, cnt_out, acc, icba, icbb, ones, isema, isemb):
        c = lax.axis_index("c")
        s = lax.axis_index("s")
        wid = c * NS + s
        r0 = s * RPT

        pltpu.sync_copy(zrow_hbm.at[pl.ds(r0, RPT)], acc.at[pl.ds(r0, RPT)])

        @pl.loop(0, WIN)
        def _(i):
            @pl.loop(0, D, step=16)
            def _(j):
                ones[i, pl.ds(j, 16)] = jnp.full((16,), 1.0, jnp.float32)

        pltpu.sync_copy(idx_hbm.at[wid, 0], icba)
        pltpu.async_copy(idx_hbm.at[wid, 1], icbb, isemb)
        plsc.subcore_barrier()

        @pl.loop(0, NITER)
        def _(k):
            ca = 2 * k
            pltpu.sync_copy(ones, acc.at[icba.at[0, 1]], add=True)
            pltpu.sync_copy(ones, acc.at[icba.at[1, 1]], add=True)
            pltpu.sync_copy(ones, acc.at[icba.at[2, 1]], add=True)
            pltpu.sync_copy(ones, acc.at[icba.at[3, 1]], add=True)
            pltpu.async_copy(idx_hbm.at[wid, ca + 2], icba, isema)
            pltpu.make_async_copy(idx_hbm.at[wid, 0], icbb, isemb).wait()
            pltpu.sync_copy(ones, acc.at[icbb.at[0, 1]], add=True)
            pltpu.sync_copy(ones, acc.at[icbb.at[1, 1]], add=True)
            pltpu.sync_copy(ones, acc.at[icbb.at[2, 1]], add=True)
            pltpu.sync_copy(ones, acc.at[icbb.at[3, 1]], add=True)
            pltpu.async_copy(idx_hbm.at[wid, ca + 3], icbb, isemb)
            pltpu.make_async_copy(idx_hbm.at[wid, 0], icba, isema).wait()

        pltpu.make_async_copy(idx_hbm.at[wid, 0], icbb, isemb).wait()
        plsc.subcore_barrier()
        pltpu.sync_copy(acc.at[pl.ds(r0, RPT)], cnt_out.at[c].at[pl.ds(r0, RPT)])

    return pl.kernel(body, out_type=out, mesh=mesh, scratch_types=scratch)


_sc_agg = _sc_agg_kernel()
_sc_cnt = _sc_cnt_kernel()


def _dotT(a, w):
    return lax.dot_general(a, w, (((1,), (1,)), ((), ())),
                           preferred_element_type=jnp.float32)


def _tc_layer1(p, cc, x, wl, b, wr, h_out):
    cnt = jnp.maximum(cc[0][:, 0:1] + cc[1][:, 0:1], 1.0)
    agg = (p[0] + p[1]) / cnt
    h = _dotT(agg, wl[...]) + _dotT(x[...], wr[...]) + b[...]
    h_out[...] = jnp.maximum(h, 0.0)


def _tc_layer2(p, cc, h1, wl, b, wr, wc, bc, o_out):
    cnt = jnp.maximum(cc[0][:, 0:1] + cc[1][:, 0:1], 1.0)
    agg = (p[0] + p[1]) / cnt
    h = _dotT(agg, wl[...]) + _dotT(h1[...], wr[...]) + b[...]
    h = jnp.maximum(h, 0.0)
    logit = jnp.sum(h * wc[...], axis=1, keepdims=True) + bc[...]
    o_out[...] = jax.nn.sigmoid(logit)


_BLK = 1000
_GRID = N // _BLK


def _rows(i):
    return (i, 0)


_p_spec = pl.BlockSpec((2, _BLK, D), lambda i: (0, i, 0))
_x_spec = pl.BlockSpec((_BLK, D), _rows)
_w_spec = pl.BlockSpec((D, D), lambda i: (0, 0))
_b_spec = pl.BlockSpec((1, D), lambda i: (0, 0))
_s_spec = pl.BlockSpec((1, 1), lambda i: (0, 0))

_tc1 = pl.pallas_call(
    _tc_layer1,
    grid=(_GRID,),
    in_specs=[_p_spec, _p_spec, _x_spec, _w_spec, _b_spec, _w_spec],
    out_specs=_x_spec,
    out_shape=jax.ShapeDtypeStruct((N, D), jnp.float32),
)

_tc2 = pl.pallas_call(
    _tc_layer2,
    grid=(_GRID,),
    in_specs=[_p_spec, _p_spec, _x_spec, _w_spec, _b_spec, _w_spec,
              _b_spec, _s_spec],
    out_specs=pl.BlockSpec((_BLK, 1), _rows),
    out_shape=jax.ShapeDtypeStruct((N, 1), jnp.float32),
)


def kernel(x, edge_index, W1l, b1, W1r, W2l, b2, W2r, Wc, bc):
    src = edge_index[0].reshape(NW, EPW)
    dst = edge_index[1].reshape(NW, EPW)
    pad = WALLOC * WIN - EPW
    src = jnp.concatenate([src, jnp.zeros((NW, pad), jnp.int32)], axis=1)
    dst = jnp.concatenate([dst, jnp.full((NW, pad), DUMP, jnp.int32)], axis=1)
    idx = jnp.concatenate([src.reshape(NW, NCHUNK, CH, 1, WIN),
                           dst.reshape(NW, NCHUNK, CH, 1, WIN)], axis=3)
    zrow = jnp.zeros((N_PAD, D), jnp.float32)

    cnt = _sc_cnt(idx, zrow)
    p1 = _sc_agg(x, idx, zrow)
    h1 = _tc1(p1, cnt, x, W1l, b1.reshape(1, D), W1r)
    p2 = _sc_agg(h1, idx, zrow)
    out = _tc2(p2, cnt, h1, W2l, b2.reshape(1, D), W2r,
               Wc.reshape(1, D), bc.reshape(1, 1))
    return out


# speedup vs baseline: 1.4905x; 1.4905x over previous
import jax
import jax.numpy as jnp
from jax import lax
from jax.experimental import pallas as pl
from jax.experimental.pallas import tpu as pltpu
from jax.experimental.pallas import tpu_sc as plsc

N = 10000
E = 320000
D = 128

NC = 2
NS = 16
NW = NC * NS
EPW = E // NW
N_PAD = 10112
DUMP = N
RPT = N_PAD // NS

WIN = 64
NWIN = 158
WALLOC = NWIN + 2

WINC = 128
CH = 4
NITERC = 10
NWINC = 8 * NITERC
NCHUNKC = NWINC // CH + 2


def _sc_agg_kernel():
    mesh = plsc.VectorSubcoreMesh(core_axis_name="c", subcore_axis_name="s")
    out = jax.ShapeDtypeStruct((NC, N_PAD, D), jnp.float32)
    scratch = [
        pltpu.VMEM_SHARED((N_PAD, D), jnp.float32),
        pltpu.VMEM((2, WIN), jnp.int32),
        pltpu.VMEM((2, WIN), jnp.int32),
        pltpu.VMEM((WIN, D), jnp.float32),
        pltpu.VMEM((WIN, D), jnp.float32),
        pltpu.SemaphoreType.DMA,
        pltpu.SemaphoreType.DMA,
        pltpu.SemaphoreType.DMA,
        pltpu.SemaphoreType.DMA,
    ]

    def body(x_hbm, idx_hbm, zrow_hbm, agg_out, acc, idx0, idx1,
             rows0, rows1, sem0, sem1, isem0, isem1):
        c = lax.axis_index("c")
        s = lax.axis_index("s")
        wid = c * NS + s
        r0 = s * RPT

        pltpu.sync_copy(zrow_hbm.at[pl.ds(r0, RPT)], acc.at[pl.ds(r0, RPT)])
        pltpu.sync_copy(idx_hbm.at[wid, 0], idx0)
        pltpu.sync_copy(idx_hbm.at[wid, 1], idx1)
        pltpu.async_copy(x_hbm.at[idx0.at[0]], rows0, sem0)
        pltpu.async_copy(x_hbm.at[idx1.at[0]], rows1, sem1)
        plsc.subcore_barrier()

        @pl.loop(0, NWIN, step=2)
        def _(w):
            pltpu.make_async_copy(x_hbm.at[idx0.at[0]], rows0, sem0).wait()
            pltpu.sync_copy(rows0, acc.at[idx0.at[1]], add=True)
            pltpu.async_copy(idx_hbm.at[wid, w + 2], idx0, isem0)
            pltpu.make_async_copy(x_hbm.at[idx1.at[0]], rows1, sem1).wait()
            pltpu.sync_copy(rows1, acc.at[idx1.at[1]], add=True)
            pltpu.async_copy(idx_hbm.at[wid, w + 3], idx1, isem1)
            pltpu.make_async_copy(idx_hbm.at[wid, 0], idx0, isem0).wait()
            pltpu.async_copy(x_hbm.at[idx0.at[0]], rows0, sem0)
            pltpu.make_async_copy(idx_hbm.at[wid, 0], idx1, isem1).wait()
            pltpu.async_copy(x_hbm.at[idx1.at[0]], rows1, sem1)

        pltpu.make_async_copy(x_hbm.at[idx0.at[0]], rows0, sem0).wait()
        pltpu.make_async_copy(x_hbm.at[idx1.at[0]], rows1, sem1).wait()
        plsc.subcore_barrier()

        pltpu.sync_copy(acc.at[pl.ds(r0, RPT)], agg_out.at[c].at[pl.ds(r0, RPT)])

    return pl.kernel(body, out_type=out, mesh=mesh, scratch_types=scratch)


def _sc_cnt_kernel():
    mesh = plsc.VectorSubcoreMesh(core_axis_name="c", subcore_axis_name="s")
    out = jax.ShapeDtypeStruct((NC, N_PAD, D), jnp.float32)
    scratch = [
        pltpu.VMEM_SHARED((N_PAD, D), jnp.float32),
        pltpu.VMEM((CH, 2, WINC), jnp.int32),
        pltpu.VMEM((CH, 2, WINC), jnp.int32),
        pltpu.VMEM((WINC, D), jnp.float32),
        pltpu.SemaphoreType.DMA,
        pltpu.SemaphoreType.DMA,
    ]

    def body(idx_hbm, zrow_hbm, cnt_out, acc, icba, icbb, ones, isema, isemb):
        c = lax.axis_index("c")
        s = lax.axis_index("s")
        wid = c * NS + s
        r0 = s * RPT

        pltpu.sync_copy(zrow_hbm.at[pl.ds(r0, RPT)], acc.at[pl.ds(r0, RPT)])

        @pl.loop(0, WINC)
        def _(i):
            @pl.loop(0, D, step=16)
            def _(j):
                ones[i, pl.ds(j, 16)] = jnp.full((16,), 1.0, jnp.float32)

        pltpu.sync_copy(idx_hbm.at[wid, 0], icba)
        pltpu.async_copy(idx_hbm.at[wid, 1], icbb, isemb)
        plsc.subcore_barrier()

        @pl.loop(0, NITERC)
        def _(k):
            ca = 2 * k
            pltpu.sync_copy(ones, acc.at[icba.at[0, 1]], add=True)
            pltpu.sync_copy(ones, acc.at[icba.at[1, 1]], add=True)
            pltpu.sync_copy(ones, acc.at[icba.at[2, 1]], add=True)
            pltpu.sync_copy(ones, acc.at[icba.at[3, 1]], add=True)
            pltpu.async_copy(idx_hbm.at[wid, ca + 2], icba, isema)
            pltpu.make_async_copy(idx_hbm.at[wid, 0], icbb, isemb).wait()
            pltpu.sync_copy(ones, acc.at[icbb.at[0, 1]], add=True)
            pltpu.sync_copy(ones, acc.at[icbb.at[1, 1]], add=True)
            pltpu.sync_copy(ones, acc.at[icbb.at[2, 1]], add=True)
            pltpu.sync_copy(ones, acc.at[icbb.at[3, 1]], add=True)
            pltpu.async_copy(idx_hbm.at[wid, ca + 3], icbb, isemb)
            pltpu.make_async_copy(idx_hbm.at[wid, 0], icba, isema).wait()

        pltpu.make_async_copy(idx_hbm.at[wid, 0], icbb, isemb).wait()
        plsc.subcore_barrier()
        pltpu.sync_copy(acc.at[pl.ds(r0, RPT)], cnt_out.at[c].at[pl.ds(r0, RPT)])

    return pl.kernel(body, out_type=out, mesh=mesh, scratch_types=scratch)


_sc_agg = _sc_agg_kernel()
_sc_cnt = _sc_cnt_kernel()


def _dotT(a, w):
    return lax.dot_general(a, w, (((1,), (1,)), ((), ())),
                           preferred_element_type=jnp.float32)


def _tc_layer1(p, cc, x, wl, b, wr, h_out):
    cnt = jnp.maximum(cc[0][:, 0:1] + cc[1][:, 0:1], 1.0)
    agg = (p[0] + p[1]) / cnt
    h = _dotT(agg, wl[...]) + _dotT(x[...], wr[...]) + b[...]
    h_out[...] = jnp.maximum(h, 0.0)


def _tc_layer2(p, cc, h1, wl, b, wr, wc, bc, o_out):
    cnt = jnp.maximum(cc[0][:, 0:1] + cc[1][:, 0:1], 1.0)
    agg = (p[0] + p[1]) / cnt
    h = _dotT(agg, wl[...]) + _dotT(h1[...], wr[...]) + b[...]
    h = jnp.maximum(h, 0.0)
    logit = jnp.sum(h * wc[...], axis=1, keepdims=True) + bc[...]
    o_out[...] = jax.nn.sigmoid(logit)


_BLK = 1000
_GRID = N // _BLK


def _rows(i):
    return (i, 0)


_p_spec = pl.BlockSpec((2, _BLK, D), lambda i: (0, i, 0))
_x_spec = pl.BlockSpec((_BLK, D), _rows)
_w_spec = pl.BlockSpec((D, D), lambda i: (0, 0))
_b_spec = pl.BlockSpec((1, D), lambda i: (0, 0))
_s_spec = pl.BlockSpec((1, 1), lambda i: (0, 0))

_tc1 = pl.pallas_call(
    _tc_layer1,
    grid=(_GRID,),
    in_specs=[_p_spec, _p_spec, _x_spec, _w_spec, _b_spec, _w_spec],
    out_specs=_x_spec,
    out_shape=jax.ShapeDtypeStruct((N, D), jnp.float32),
)

_tc2 = pl.pallas_call(
    _tc_layer2,
    grid=(_GRID,),
    in_specs=[_p_spec, _p_spec, _x_spec, _w_spec, _b_spec, _w_spec,
              _b_spec, _s_spec],
    out_specs=pl.BlockSpec((_BLK, 1), _rows),
    out_shape=jax.ShapeDtypeStruct((N, 1), jnp.float32),
)


def kernel(x, edge_index, W1l, b1, W1r, W2l, b2, W2r, Wc, bc):
    src = edge_index[0].reshape(NW, EPW)
    dst = edge_index[1].reshape(NW, EPW)

    pad = WALLOC * WIN - EPW
    srcf = jnp.concatenate([src, jnp.zeros((NW, pad), jnp.int32)], axis=1)
    dstf = jnp.concatenate([dst, jnp.full((NW, pad), DUMP, jnp.int32)], axis=1)
    idxf = jnp.stack([srcf.reshape(NW, WALLOC, WIN),
                      dstf.reshape(NW, WALLOC, WIN)], axis=2)

    padc = NCHUNKC * CH * WINC - EPW
    srcc = jnp.concatenate([src, jnp.zeros((NW, padc), jnp.int32)], axis=1)
    dstc = jnp.concatenate([dst, jnp.full((NW, padc), DUMP, jnp.int32)], axis=1)
    idxc = jnp.concatenate([srcc.reshape(NW, NCHUNKC, CH, 1, WINC),
                            dstc.reshape(NW, NCHUNKC, CH, 1, WINC)], axis=3)

    zrow = jnp.zeros((N_PAD, D), jnp.float32)

    cnt = _sc_cnt(idxc, zrow)
    p1 = _sc_agg(x, idxf, zrow)
    h1 = _tc1(p1, cnt, x, W1l, b1.reshape(1, D), W1r)
    p2 = _sc_agg(h1, idxf, zrow)
    out = _tc2(p2, cnt, h1, W2l, b2.reshape(1, D), W2r,
               Wc.reshape(1, D), bc.reshape(1, 1))
    return out
